# baseline (speedup 1.0000x reference)
import dataclasses
import functools

import jax
import jax.numpy as jnp
from jax import lax
from jax.experimental import pallas as pl
from jax.experimental.pallas import tpu as pltpu
from jax.experimental.pallas import tpu_sc as plsc

_NC = 2
_NS = 16
_L = 16
_NBUF = 4
_GRP = 16


def _fm_forward(idx, bias1d, vect_weight):
    B, F = idx.shape
    D = vect_weight.shape[1]
    NW = _NC * _NS
    BW = B // NW
    DK = D // _L
    FC = F // _L
    TAIL = F - FC * _L
    FP = ((F + _L - 1) // _L) * _L

    mesh = plsc.VectorSubcoreMesh(core_axis_name="c", subcore_axis_name="s")
    scratch = (
        [pltpu.VMEM((BW, F), jnp.int32)]
        + [pltpu.VMEM((F, D), jnp.float32) for _ in range(_NBUF)]
        + [pltpu.VMEM((FP,), jnp.float32) for _ in range(_NBUF)]
        + [
            pltpu.VMEM((D,), jnp.float32),
            pltpu.VMEM((_L,), jnp.float32),
            pltpu.VMEM((BW,), jnp.float32),
        ]
        + [pltpu.SemaphoreType.DMA for _ in range(_NBUF)]
    )

    cp = pltpu.CompilerParams()
    if "needs_layout_passes" in pltpu.CompilerParams.__dataclass_fields__:
        cp = dataclasses.replace(cp, needs_layout_passes=False)

    @functools.partial(
        pl.kernel,
        out_type=jax.ShapeDtypeStruct((B,), jnp.float32),
        mesh=mesh,
        scratch_types=scratch,
        compiler_params=cp,
    )
    def fm_kernel(idx_hbm, bias_hbm, vect_hbm, out_hbm, *scr):
        idx_v = scr[0]
        rows = scr[1 : 1 + _NBUF]
        bbuf = scr[1 + _NBUF : 1 + 2 * _NBUF]
        v0_v, b16_v, out_v = scr[1 + 2 * _NBUF : 4 + 2 * _NBUF]
        sems = scr[4 + 2 * _NBUF :]

        wid = lax.axis_index("s") * _NC + lax.axis_index("c")
        base = wid * BW

        pltpu.sync_copy(idx_hbm.at[pl.ds(base, BW)], idx_v)
        pltpu.sync_copy(vect_hbm.at[0], v0_v)
        pltpu.sync_copy(bias_hbm.at[pl.ds(0, _L)], b16_v)

        zeros = jnp.zeros((_L,), jnp.float32)
        if TAIL:
            for j in range(_NBUF):
                bbuf[j][pl.ds(FP - _L, _L)] = zeros

        lanes = lax.iota(jnp.int32, _L)
        v0c = [v0_v[pl.ds(k * _L, _L)] for k in range(DK)]
        q0v = functools.reduce(lambda a, b: a + b, [c * c for c in v0c])
        b0v = jnp.full((_L,), b16_v[pl.ds(0, _L)][0])

        def start(b, j):
            pltpu.make_async_copy(vect_hbm.at[idx_v.at[b]], rows[j], sems[j]).start()
            pltpu.make_async_copy(
                bias_hbm.at[idx_v.at[b]], bbuf[j].at[pl.ds(0, F)], sems[j]
            ).start()

        def wait(b, j):
            pltpu.make_async_copy(vect_hbm.at[idx_v.at[b]], rows[j], sems[j]).wait()
            pltpu.make_async_copy(
                bias_hbm.at[idx_v.at[b]], bbuf[j].at[pl.ds(0, F)], sems[j]
            ).wait()

        for j in range(_NBUF):
            start(j, j)

        @pl.loop(0, BW, step=_GRP)
        def _(g):
            out_acc = zeros
            for j in range(_GRP):
                b = g + j
                buf = j % _NBUF
                wait(b, buf)

                cnt = zeros
                for c in range(FC):
                    ch = idx_v[b, pl.ds(c * _L, _L)]
                    cnt = cnt + jnp.where(ch == 0, 1.0, 0.0).astype(jnp.float32)
                if TAIL:
                    ch = idx_v[b, pl.ds(F - _L, _L)]
                    sel = (ch == 0) & (lanes >= (_L - TAIL))
                    cnt = cnt + jnp.where(sel, 1.0, 0.0).astype(jnp.float32)
                c0v = jnp.full((_L,), jnp.sum(cnt))

                bvec = zeros
                for c in range(FP // _L):
                    bvec = bvec + bbuf[buf][pl.ds(c * _L, _L)]

                def body(f, carry):
                    accs = carry[:DK]
                    qs = carry[DK:]
                    new_s, new_q = [], []
                    for k in range(DK):
                        v = rows[buf][f, pl.ds(k * _L, _L)]
                        new_s.append(accs[k] + v)
                        new_q.append(qs[k] + v * v)
                    return tuple(new_s) + tuple(new_q)

                init = (zeros,) * (2 * DK)
                res = lax.fori_loop(0, F, body, init, unroll=10)

                smsq = zeros
                qtot = zeros
                for k in range(DK):
                    sm = res[k] - c0v * v0c[k]
                    smsq = smsq + sm * sm
                    qtot = qtot + res[DK + k]
                qcorr = qtot - c0v * q0v
                rvec = bvec - cnt * b0v + 0.5 * (smsq - qcorr)
                r = jnp.sum(rvec)
                out_acc = jnp.where(lanes == j, jnp.full((_L,), r), out_acc)

                @pl.when(b + _NBUF < BW)
                def _():
                    start(b + _NBUF, buf)

            out_v[pl.ds(g, _GRP)] = out_acc

        pltpu.sync_copy(out_v, out_hbm.at[pl.ds(base, BW)])

    return fm_kernel(idx, bias1d, vect_weight)


def kernel(idx, bias_weight, vect_weight):
    idx = idx.astype(jnp.int32)
    bias1d = bias_weight.reshape(-1)
    return _fm_forward(idx, bias1d, vect_weight)


# speedup vs baseline: 1.1161x; 1.1161x over previous
import dataclasses
import functools

import jax
import jax.numpy as jnp
from jax import lax
from jax.experimental import pallas as pl
from jax.experimental.pallas import tpu as pltpu
from jax.experimental.pallas import tpu_sc as plsc

_NC = 2
_NS = 16
_L = 16
_NBUF = 4
_GRP = 16


def _fm_forward(idx, bias1d, vect_weight):
    B, F = idx.shape
    D = vect_weight.shape[1]
    NW = _NC * _NS
    BW = B // NW
    DK = D // _L
    FC = F // _L
    TAIL = F - FC * _L
    FP = ((F + _L - 1) // _L) * _L

    mesh = plsc.VectorSubcoreMesh(core_axis_name="c", subcore_axis_name="s")
    scratch = (
        [pltpu.VMEM((BW, F), jnp.int32)]
        + [pltpu.VMEM((F, D), jnp.float32) for _ in range(_NBUF)]
        + [pltpu.VMEM((FP,), jnp.float32) for _ in range(_NBUF)]
        + [
            pltpu.VMEM((D,), jnp.float32),
            pltpu.VMEM((_L,), jnp.float32),
            pltpu.VMEM((BW,), jnp.float32),
        ]
        + [pltpu.SemaphoreType.DMA for _ in range(_NBUF)]
    )

    cp = pltpu.CompilerParams()
    if "needs_layout_passes" in pltpu.CompilerParams.__dataclass_fields__:
        cp = dataclasses.replace(cp, needs_layout_passes=False)

    @functools.partial(
        pl.kernel,
        out_type=jax.ShapeDtypeStruct((B,), jnp.float32),
        mesh=mesh,
        scratch_types=scratch,
        compiler_params=cp,
    )
    def fm_kernel(idx_hbm, bias_hbm, vect_hbm, out_hbm, *scr):
        idx_v = scr[0]
        rows = scr[1 : 1 + _NBUF]
        bbuf = scr[1 + _NBUF : 1 + 2 * _NBUF]
        v0_v, b16_v, out_v = scr[1 + 2 * _NBUF : 4 + 2 * _NBUF]
        sems = scr[4 + 2 * _NBUF :]

        wid = lax.axis_index("s") * _NC + lax.axis_index("c")
        base = wid * BW

        pltpu.sync_copy(idx_hbm.at[pl.ds(base, BW)], idx_v)
        pltpu.sync_copy(vect_hbm.at[0], v0_v)
        pltpu.sync_copy(bias_hbm.at[pl.ds(0, _L)], b16_v)

        zeros = jnp.zeros((_L,), jnp.float32)
        if TAIL:
            for j in range(_NBUF):
                bbuf[j][pl.ds(FP - _L, _L)] = zeros

        lanes = lax.iota(jnp.int32, _L)
        v0c = [v0_v[pl.ds(k * _L, _L)] for k in range(DK)]
        q0v = functools.reduce(lambda a, b: a + b, [c * c for c in v0c])
        b0v = jnp.full((_L,), b16_v[pl.ds(0, _L)][0])

        def start(b, j):
            pltpu.make_async_copy(vect_hbm.at[idx_v.at[b]], rows[j], sems[j]).start()
            pltpu.make_async_copy(
                bias_hbm.at[idx_v.at[b]], bbuf[j].at[pl.ds(0, F)], sems[j]
            ).start()

        def wait(b, j):
            pltpu.make_async_copy(vect_hbm.at[idx_v.at[b]], rows[j], sems[j]).wait()
            pltpu.make_async_copy(
                bias_hbm.at[idx_v.at[b]], bbuf[j].at[pl.ds(0, F)], sems[j]
            ).wait()

        for j in range(_NBUF):
            start(j, j)

        @pl.loop(0, BW, step=_GRP)
        def _(g):
            out_acc = zeros
            for j in range(_GRP):
                b = g + j
                buf = j % _NBUF
                wait(b, buf)

                cnt = zeros
                for c in range(FC):
                    ch = idx_v[b, pl.ds(c * _L, _L)]
                    cnt = cnt + jnp.where(ch == 0, 1.0, 0.0).astype(jnp.float32)
                if TAIL:
                    ch = idx_v[b, pl.ds(F - _L, _L)]
                    sel = (ch == 0) & (lanes >= (_L - TAIL))
                    cnt = cnt + jnp.where(sel, 1.0, 0.0).astype(jnp.float32)
                c0v = jnp.full((_L,), jnp.sum(cnt))

                bvec = zeros
                for c in range(FP // _L):
                    bvec = bvec + bbuf[buf][pl.ds(c * _L, _L)]

                def body(f, carry):
                    accs = carry[:DK]
                    qs = carry[DK:]
                    new_s, new_q = [], []
                    for k in range(DK):
                        v = rows[buf][f, pl.ds(k * _L, _L)]
                        new_s.append(accs[k] + v)
                        new_q.append(qs[k] + v * v)
                    return tuple(new_s) + tuple(new_q)

                init = (zeros,) * (2 * DK)
                res = lax.fori_loop(0, F, body, init, unroll=4)

                smsq = zeros
                qtot = zeros
                for k in range(DK):
                    sm = res[k] - c0v * v0c[k]
                    smsq = smsq + sm * sm
                    qtot = qtot + res[DK + k]
                qcorr = qtot - c0v * q0v
                rvec = bvec - cnt * b0v + 0.5 * (smsq - qcorr)
                r = jnp.sum(rvec)
                out_acc = jnp.where(lanes == j, jnp.full((_L,), r), out_acc)

                @pl.when(b + _NBUF < BW)
                def _():
                    start(b + _NBUF, buf)

            out_v[pl.ds(g, _GRP)] = out_acc

        pltpu.sync_copy(out_v, out_hbm.at[pl.ds(base, BW)])

    return fm_kernel(idx, bias1d, vect_weight)


def kernel(idx, bias_weight, vect_weight):
    idx = idx.astype(jnp.int32)
    bias1d = bias_weight.reshape(-1)
    return _fm_forward(idx, bias1d, vect_weight)


# speedup vs baseline: 1.2050x; 1.0796x over previous
import dataclasses
import functools

import jax
import jax.numpy as jnp
from jax import lax
from jax.experimental import pallas as pl
from jax.experimental.pallas import tpu as pltpu
from jax.experimental.pallas import tpu_sc as plsc

_NC = 2
_NS = 16
_L = 16
_NBUF = 8
_GRP = 16


def _fm_forward(idx, bias1d, vect_weight):
    B, F = idx.shape
    D = vect_weight.shape[1]
    NW = _NC * _NS
    BW = B // NW
    DK = D // _L
    FC = F // _L
    TAIL = F - FC * _L
    FP = ((F + _L - 1) // _L) * _L

    mesh = plsc.VectorSubcoreMesh(core_axis_name="c", subcore_axis_name="s")
    scratch = (
        [pltpu.VMEM((BW, F), jnp.int32)]
        + [pltpu.VMEM((F, D), jnp.float32) for _ in range(_NBUF)]
        + [pltpu.VMEM((FP,), jnp.float32) for _ in range(_NBUF)]
        + [
            pltpu.VMEM((D,), jnp.float32),
            pltpu.VMEM((_L,), jnp.float32),
            pltpu.VMEM((BW,), jnp.float32),
        ]
        + [pltpu.SemaphoreType.DMA for _ in range(_NBUF)]
    )

    cp = pltpu.CompilerParams()
    if "needs_layout_passes" in pltpu.CompilerParams.__dataclass_fields__:
        cp = dataclasses.replace(cp, needs_layout_passes=False)

    @functools.partial(
        pl.kernel,
        out_type=jax.ShapeDtypeStruct((B,), jnp.float32),
        mesh=mesh,
        scratch_types=scratch,
        compiler_params=cp,
    )
    def fm_kernel(idx_hbm, bias_hbm, vect_hbm, out_hbm, *scr):
        idx_v = scr[0]
        rows = scr[1 : 1 + _NBUF]
        bbuf = scr[1 + _NBUF : 1 + 2 * _NBUF]
        v0_v, b16_v, out_v = scr[1 + 2 * _NBUF : 4 + 2 * _NBUF]
        sems = scr[4 + 2 * _NBUF :]

        wid = lax.axis_index("s") * _NC + lax.axis_index("c")
        base = wid * BW

        pltpu.sync_copy(idx_hbm.at[pl.ds(base, BW)], idx_v)
        pltpu.sync_copy(vect_hbm.at[0], v0_v)
        pltpu.sync_copy(bias_hbm.at[pl.ds(0, _L)], b16_v)

        zeros = jnp.zeros((_L,), jnp.float32)
        if TAIL:
            for j in range(_NBUF):
                bbuf[j][pl.ds(FP - _L, _L)] = zeros

        lanes = lax.iota(jnp.int32, _L)
        v0c = [v0_v[pl.ds(k * _L, _L)] for k in range(DK)]
        q0v = functools.reduce(lambda a, b: a + b, [c * c for c in v0c])
        b0v = jnp.full((_L,), b16_v[pl.ds(0, _L)][0])

        def start(b, j):
            pltpu.make_async_copy(vect_hbm.at[idx_v.at[b]], rows[j], sems[j]).start()
            pltpu.make_async_copy(
                bias_hbm.at[idx_v.at[b]], bbuf[j].at[pl.ds(0, F)], sems[j]
            ).start()

        def wait(b, j):
            pltpu.make_async_copy(vect_hbm.at[idx_v.at[b]], rows[j], sems[j]).wait()
            pltpu.make_async_copy(
                bias_hbm.at[idx_v.at[b]], bbuf[j].at[pl.ds(0, F)], sems[j]
            ).wait()

        for j in range(_NBUF):
            start(j, j)

        @pl.loop(0, BW, step=_GRP)
        def _(g):
            out_acc = zeros
            for j in range(_GRP):
                b = g + j
                buf = j % _NBUF
                wait(b, buf)

                cnt = zeros
                for c in range(FC):
                    ch = idx_v[b, pl.ds(c * _L, _L)]
                    cnt = cnt + jnp.where(ch == 0, 1.0, 0.0).astype(jnp.float32)
                if TAIL:
                    ch = idx_v[b, pl.ds(F - _L, _L)]
                    sel = (ch == 0) & (lanes >= (_L - TAIL))
                    cnt = cnt + jnp.where(sel, 1.0, 0.0).astype(jnp.float32)
                c0v = jnp.full((_L,), jnp.sum(cnt))

                bvec = zeros
                for c in range(FP // _L):
                    bvec = bvec + bbuf[buf][pl.ds(c * _L, _L)]

                def body(f, carry):
                    accs = carry[:DK]
                    qs = carry[DK:]
                    new_s, new_q = [], []
                    for k in range(DK):
                        v = rows[buf][f, pl.ds(k * _L, _L)]
                        new_s.append(accs[k] + v)
                        new_q.append(qs[k] + v * v)
                    return tuple(new_s) + tuple(new_q)

                init = (zeros,) * (2 * DK)
                res = lax.fori_loop(0, F, body, init, unroll=4)

                smsq = zeros
                qtot = zeros
                for k in range(DK):
                    sm = res[k] - c0v * v0c[k]
                    smsq = smsq + sm * sm
                    qtot = qtot + res[DK + k]
                qcorr = qtot - c0v * q0v
                rvec = bvec - cnt * b0v + 0.5 * (smsq - qcorr)
                r = jnp.sum(rvec)
                out_acc = jnp.where(lanes == j, jnp.full((_L,), r), out_acc)

                @pl.when(b + _NBUF < BW)
                def _():
                    start(b + _NBUF, buf)

            out_v[pl.ds(g, _GRP)] = out_acc

        pltpu.sync_copy(out_v, out_hbm.at[pl.ds(base, BW)])

    return fm_kernel(idx, bias1d, vect_weight)


def kernel(idx, bias_weight, vect_weight):
    idx = idx.astype(jnp.int32)
    bias1d = bias_weight.reshape(-1)
    return _fm_forward(idx, bias1d, vect_weight)
